# baseline (speedup 1.0000x reference)
import jax
import jax.numpy as jnp
from jax import lax
from jax.experimental import pallas as pl
from jax.experimental.pallas import tpu as pltpu
from jax.experimental.pallas import tpu_sc as plsc

N = 10000
D = 128
NC = 2
NS = 16
CH = 128
K0 = 126
K1 = 31
KM = max(K0, K1)
E0 = NS * K0 * CH
E1CAP = NS * K1 * CH
NPAD = 10112
RPT = NPAD // NS
DEGW = 16

_mesh = plsc.VectorSubcoreMesh(core_axis_name="c", subcore_axis_name="s",
                               num_cores=NC, num_subcores=NS)


def _deg_body(dst_hbm, degw_hbm, dstv, onesv, zv, acc):
    cid = lax.axis_index("c")
    tid = lax.axis_index("s")
    pltpu.sync_copy(dst_hbm.at[cid, tid], dstv)

    def fill(i, _):
        onesv[i, :] = jnp.full((DEGW,), 1.0, jnp.float32)
        zv[i, :] = jnp.zeros((DEGW,), jnp.float32)
        return 0

    lax.fori_loop(0, CH, fill, 0)
    base = tid * RPT
    for r0 in range(0, RPT, CH):
        sz = min(CH, RPT - r0)
        pltpu.sync_copy(zv.at[pl.ds(0, sz)], acc.at[pl.ds(base + r0, sz)])
    plsc.subcore_barrier()

    def chunk(j, _):
        pltpu.sync_copy(onesv, acc.at[dstv.at[j]], add=True)
        return 0

    myk = jnp.where(cid == 0, K0, K1)
    lax.fori_loop(0, myk, chunk, 0)
    plsc.subcore_barrier()
    pltpu.sync_copy(acc.at[pl.ds(base, RPT)], degw_hbm.at[cid, pl.ds(base, RPT)])


_deg_call = pl.kernel(
    _deg_body,
    out_type=jax.ShapeDtypeStruct((NC, NPAD, DEGW), jnp.float32),
    mesh=_mesh,
    scratch_types=[
        pltpu.VMEM((KM, CH), jnp.int32),
        pltpu.VMEM((CH, DEGW), jnp.float32),
        pltpu.VMEM((CH, DEGW), jnp.float32),
        pltpu.VMEM_SHARED((NPAD, DEGW), jnp.float32),
    ],
)


def _scat_body(g_hbm, src_hbm, dst_hbm, s_hbm, srcv, dstv, rows, acc):
    cid = lax.axis_index("c")
    tid = lax.axis_index("s")
    pltpu.sync_copy(src_hbm.at[cid, tid], srcv)
    pltpu.sync_copy(dst_hbm.at[cid, tid], dstv)

    def zb(k, _):
        rows[k // 8, pl.ds((k % 8) * 16, 16)] = jnp.zeros((16,), jnp.float32)
        return 0

    lax.fori_loop(0, CH * D // 16, zb, 0)
    base = tid * RPT
    for r0 in range(0, RPT, CH):
        sz = min(CH, RPT - r0)
        pltpu.sync_copy(rows.at[pl.ds(0, sz)], acc.at[pl.ds(base + r0, sz)])
    plsc.subcore_barrier()

    def chunk(j, _):
        pltpu.sync_copy(g_hbm.at[srcv.at[j]], rows)
        pltpu.sync_copy(rows, acc.at[dstv.at[j]], add=True)
        return 0

    myk = jnp.where(cid == 0, K0, K1)
    lax.fori_loop(0, myk, chunk, 0)
    plsc.subcore_barrier()
    pltpu.sync_copy(acc.at[pl.ds(base, RPT)], s_hbm.at[cid, pl.ds(base, RPT)])


_scat_call = pl.kernel(
    _scat_body,
    out_type=jax.ShapeDtypeStruct((NC, NPAD, D), jnp.float32),
    mesh=_mesh,
    scratch_types=[
        pltpu.VMEM((KM, CH), jnp.int32),
        pltpu.VMEM((KM, CH), jnp.int32),
        pltpu.VMEM((CH, D), jnp.float32),
        pltpu.VMEM_SHARED((NPAD, D), jnp.float32),
    ],
)


def _pre_body(degw_ref, x_ref, w_ref, dinv_ref, g_ref):
    t = degw_ref[...]
    deg = t[0, 0:N, 0:1] + t[1, 0:N, 0:1] + 1.0
    dinv = 1.0 / jnp.sqrt(deg)
    dinv_ref[...] = dinv
    g_ref[...] = dinv * jnp.dot(x_ref[...], w_ref[...],
                                preferred_element_type=jnp.float32)


def _mid_body(s_ref, g_ref, dinv_ref, b_ref, gm_ref, bt_ref, w_ref, h_ref,
              gn_ref, hp_ref=None):
    g = g_ref[...]
    s = s_ref[...]
    dinv = dinv_ref[...]
    o = dinv * (s[0, 0:N, :] + s[1, 0:N, :] + g) + b_ref[...]
    mu = jnp.mean(o, axis=0, keepdims=True)
    xc = o - mu
    var = jnp.mean(xc * xc, axis=0, keepdims=True)
    bn = gm_ref[...] * xc / jnp.sqrt(var + 1e-5) + bt_ref[...]
    h = jnp.maximum(bn, 0.0)
    if hp_ref is not None:
        h = hp_ref[...] + h
    h_ref[...] = h
    gn_ref[...] = dinv * jnp.dot(h, w_ref[...],
                                 preferred_element_type=jnp.float32)


def _mid_body_resid(s_ref, g_ref, dinv_ref, b_ref, gm_ref, bt_ref, hp_ref,
                    w_ref, h_ref, gn_ref):
    _mid_body(s_ref, g_ref, dinv_ref, b_ref, gm_ref, bt_ref, w_ref, h_ref,
              gn_ref, hp_ref=hp_ref)


def _fin_body(s_ref, g_ref, dinv_ref, b_ref, gm_ref, bt_ref, hp_ref, out_ref):
    g = g_ref[...]
    s = s_ref[...]
    o = dinv_ref[...] * (s[0, 0:N, :] + s[1, 0:N, :] + g) + b_ref[...]
    mu = jnp.mean(o, axis=0, keepdims=True)
    xc = o - mu
    var = jnp.mean(xc * xc, axis=0, keepdims=True)
    bn = gm_ref[...] * xc / jnp.sqrt(var + 1e-5) + bt_ref[...]
    out_ref[...] = hp_ref[...] + bn


_pre_call = pl.pallas_call(
    _pre_body,
    out_shape=[jax.ShapeDtypeStruct((N, 1), jnp.float32),
               jax.ShapeDtypeStruct((N, D), jnp.float32)],
)

_mid_call0 = pl.pallas_call(
    _mid_body,
    out_shape=[jax.ShapeDtypeStruct((N, D), jnp.float32),
               jax.ShapeDtypeStruct((N, D), jnp.float32)],
)

_mid_call1 = pl.pallas_call(
    _mid_body_resid,
    out_shape=[jax.ShapeDtypeStruct((N, D), jnp.float32),
               jax.ShapeDtypeStruct((N, D), jnp.float32)],
)

_fin_call = pl.pallas_call(
    _fin_body,
    out_shape=jax.ShapeDtypeStruct((N, D), jnp.float32),
)


def kernel(x, edge_index, W0, b0, gamma0, beta0, W1, b1, gamma1, beta1,
           W2, b2, gamma2, beta2):
    src = edge_index[0]
    dst = edge_index[1]
    e = src.shape[0]
    pad1 = E1CAP - (e - E0)
    padm = NS * (KM - K1) * CH
    sp = NPAD - N
    dpad1 = N + (jnp.arange(pad1, dtype=jnp.int32) % sp)
    dpadm = N + (jnp.arange(padm, dtype=jnp.int32) % sp)
    src0 = src[:E0].reshape(NS, K0, CH)
    dst0 = dst[:E0].reshape(NS, K0, CH)
    src1 = jnp.concatenate(
        [jnp.concatenate([src[E0:], jnp.zeros((pad1,), jnp.int32)]
                         ).reshape(NS, K1, CH),
         jnp.zeros((padm,), jnp.int32).reshape(NS, KM - K1, CH)], axis=1)
    dst1 = jnp.concatenate(
        [jnp.concatenate([dst[E0:], dpad1]).reshape(NS, K1, CH),
         dpadm.reshape(NS, KM - K1, CH)], axis=1)
    src4 = jnp.stack([src0, src1])
    dst4 = jnp.stack([dst0, dst1])
    b0r, b1r, b2r = (v.reshape(1, D) for v in (b0, b1, b2))
    gm0, gm1, gm2 = (v.reshape(1, D) for v in (gamma0, gamma1, gamma2))
    bt0, bt1, bt2 = (v.reshape(1, D) for v in (beta0, beta1, beta2))

    degw = _deg_call(dst4)
    dinv, g0 = _pre_call(degw, x, W0)
    s0 = _scat_call(g0, src4, dst4)
    h1, g1 = _mid_call0(s0, g0, dinv, b0r, gm0, bt0, W1)
    s1 = _scat_call(g1, src4, dst4)
    h2, g2 = _mid_call1(s1, g1, dinv, b1r, gm1, bt1, h1, W2)
    s2 = _scat_call(g2, src4, dst4)
    return _fin_call(s2, g2, dinv, b2r, gm2, bt2, h2)


# speedup vs baseline: 1.1485x; 1.1485x over previous
import jax
import jax.numpy as jnp
from jax import lax
from jax.experimental import pallas as pl
from jax.experimental.pallas import tpu as pltpu
from jax.experimental.pallas import tpu_sc as plsc

N = 10000
D = 128
NC = 2
NS = 16
CH = 128
K0 = 106
K1 = 51
KM = max(K0, K1)
E0 = NS * K0 * CH
E1CAP = NS * K1 * CH
NPAD = 10112
RPT = NPAD // NS
DEGW = 16

_mesh = plsc.VectorSubcoreMesh(core_axis_name="c", subcore_axis_name="s",
                               num_cores=NC, num_subcores=NS)


def _deg_body(dst_hbm, degw_hbm, dstv, onesv, zv, acc):
    cid = lax.axis_index("c")
    tid = lax.axis_index("s")
    pltpu.sync_copy(dst_hbm.at[cid, tid], dstv)

    def fill(i, _):
        onesv[i, :] = jnp.full((DEGW,), 1.0, jnp.float32)
        zv[i, :] = jnp.zeros((DEGW,), jnp.float32)
        return 0

    lax.fori_loop(0, CH, fill, 0)
    base = tid * RPT
    for r0 in range(0, RPT, CH):
        sz = min(CH, RPT - r0)
        pltpu.sync_copy(zv.at[pl.ds(0, sz)], acc.at[pl.ds(base + r0, sz)])
    plsc.subcore_barrier()

    def chunk(j, _):
        pltpu.sync_copy(onesv, acc.at[dstv.at[j]], add=True)
        return 0

    myk = jnp.where(cid == 0, K0, K1)
    lax.fori_loop(0, myk, chunk, 0)
    plsc.subcore_barrier()
    pltpu.sync_copy(acc.at[pl.ds(base, RPT)], degw_hbm.at[cid, pl.ds(base, RPT)])


_deg_call = pl.kernel(
    _deg_body,
    out_type=jax.ShapeDtypeStruct((NC, NPAD, DEGW), jnp.float32),
    mesh=_mesh,
    scratch_types=[
        pltpu.VMEM((KM, CH), jnp.int32),
        pltpu.VMEM((CH, DEGW), jnp.float32),
        pltpu.VMEM((CH, DEGW), jnp.float32),
        pltpu.VMEM_SHARED((NPAD, DEGW), jnp.float32),
    ],
)


def _scat_body(g_hbm, src_hbm, dst_hbm, s_hbm, srcv, dstv, rows, acc):
    cid = lax.axis_index("c")
    tid = lax.axis_index("s")
    pltpu.sync_copy(src_hbm.at[cid, tid], srcv)
    pltpu.sync_copy(dst_hbm.at[cid, tid], dstv)

    def zb(k, _):
        rows[k // 8, pl.ds((k % 8) * 16, 16)] = jnp.zeros((16,), jnp.float32)
        return 0

    lax.fori_loop(0, CH * D // 16, zb, 0)
    base = tid * RPT
    for r0 in range(0, RPT, CH):
        sz = min(CH, RPT - r0)
        pltpu.sync_copy(rows.at[pl.ds(0, sz)], acc.at[pl.ds(base + r0, sz)])
    plsc.subcore_barrier()

    def chunk(j, _):
        pltpu.sync_copy(g_hbm.at[srcv.at[j]], rows)
        pltpu.sync_copy(rows, acc.at[dstv.at[j]], add=True)
        return 0

    myk = jnp.where(cid == 0, K0, K1)
    lax.fori_loop(0, myk, chunk, 0)
    plsc.subcore_barrier()
    pltpu.sync_copy(acc.at[pl.ds(base, RPT)], s_hbm.at[cid, pl.ds(base, RPT)])


_scat_call = pl.kernel(
    _scat_body,
    out_type=jax.ShapeDtypeStruct((NC, NPAD, D), jnp.float32),
    mesh=_mesh,
    scratch_types=[
        pltpu.VMEM((KM, CH), jnp.int32),
        pltpu.VMEM((KM, CH), jnp.int32),
        pltpu.VMEM((CH, D), jnp.float32),
        pltpu.VMEM_SHARED((NPAD, D), jnp.float32),
    ],
)


def _pre_body(degw_ref, x_ref, w_ref, dinv_ref, g_ref):
    t = degw_ref[...]
    deg = t[0, 0:N, 0:1] + t[1, 0:N, 0:1] + 1.0
    dinv = 1.0 / jnp.sqrt(deg)
    dinv_ref[...] = dinv
    g_ref[...] = dinv * jnp.dot(x_ref[...], w_ref[...],
                                preferred_element_type=jnp.float32)


def _mid_body(s_ref, g_ref, dinv_ref, b_ref, gm_ref, bt_ref, w_ref, h_ref,
              gn_ref, hp_ref=None):
    g = g_ref[...]
    s = s_ref[...]
    dinv = dinv_ref[...]
    o = dinv * (s[0, 0:N, :] + s[1, 0:N, :] + g) + b_ref[...]
    mu = jnp.mean(o, axis=0, keepdims=True)
    xc = o - mu
    var = jnp.mean(xc * xc, axis=0, keepdims=True)
    bn = gm_ref[...] * xc / jnp.sqrt(var + 1e-5) + bt_ref[...]
    h = jnp.maximum(bn, 0.0)
    if hp_ref is not None:
        h = hp_ref[...] + h
    h_ref[...] = h
    gn_ref[...] = dinv * jnp.dot(h, w_ref[...],
                                 preferred_element_type=jnp.float32)


def _mid_body_resid(s_ref, g_ref, dinv_ref, b_ref, gm_ref, bt_ref, hp_ref,
                    w_ref, h_ref, gn_ref):
    _mid_body(s_ref, g_ref, dinv_ref, b_ref, gm_ref, bt_ref, w_ref, h_ref,
              gn_ref, hp_ref=hp_ref)


def _fin_body(s_ref, g_ref, dinv_ref, b_ref, gm_ref, bt_ref, hp_ref, out_ref):
    g = g_ref[...]
    s = s_ref[...]
    o = dinv_ref[...] * (s[0, 0:N, :] + s[1, 0:N, :] + g) + b_ref[...]
    mu = jnp.mean(o, axis=0, keepdims=True)
    xc = o - mu
    var = jnp.mean(xc * xc, axis=0, keepdims=True)
    bn = gm_ref[...] * xc / jnp.sqrt(var + 1e-5) + bt_ref[...]
    out_ref[...] = hp_ref[...] + bn


_pre_call = pl.pallas_call(
    _pre_body,
    out_shape=[jax.ShapeDtypeStruct((N, 1), jnp.float32),
               jax.ShapeDtypeStruct((N, D), jnp.float32)],
)

_mid_call0 = pl.pallas_call(
    _mid_body,
    out_shape=[jax.ShapeDtypeStruct((N, D), jnp.float32),
               jax.ShapeDtypeStruct((N, D), jnp.float32)],
)

_mid_call1 = pl.pallas_call(
    _mid_body_resid,
    out_shape=[jax.ShapeDtypeStruct((N, D), jnp.float32),
               jax.ShapeDtypeStruct((N, D), jnp.float32)],
)

_fin_call = pl.pallas_call(
    _fin_body,
    out_shape=jax.ShapeDtypeStruct((N, D), jnp.float32),
)


def kernel(x, edge_index, W0, b0, gamma0, beta0, W1, b1, gamma1, beta1,
           W2, b2, gamma2, beta2):
    src = edge_index[0]
    dst = edge_index[1]
    e = src.shape[0]
    pad1 = E1CAP - (e - E0)
    padm = NS * (KM - K1) * CH
    sp = NPAD - N
    dpad1 = N + (jnp.arange(pad1, dtype=jnp.int32) % sp)
    dpadm = N + (jnp.arange(padm, dtype=jnp.int32) % sp)
    src0 = src[:E0].reshape(NS, K0, CH)
    dst0 = dst[:E0].reshape(NS, K0, CH)
    src1 = jnp.concatenate(
        [jnp.concatenate([src[E0:], jnp.zeros((pad1,), jnp.int32)]
                         ).reshape(NS, K1, CH),
         jnp.zeros((padm,), jnp.int32).reshape(NS, KM - K1, CH)], axis=1)
    dst1 = jnp.concatenate(
        [jnp.concatenate([dst[E0:], dpad1]).reshape(NS, K1, CH),
         dpadm.reshape(NS, KM - K1, CH)], axis=1)
    src4 = jnp.stack([src0, src1])
    dst4 = jnp.stack([dst0, dst1])
    b0r, b1r, b2r = (v.reshape(1, D) for v in (b0, b1, b2))
    gm0, gm1, gm2 = (v.reshape(1, D) for v in (gamma0, gamma1, gamma2))
    bt0, bt1, bt2 = (v.reshape(1, D) for v in (beta0, beta1, beta2))

    degw = _deg_call(dst4)
    dinv, g0 = _pre_call(degw, x, W0)
    s0 = _scat_call(g0, src4, dst4)
    h1, g1 = _mid_call0(s0, g0, dinv, b0r, gm0, bt0, W1)
    s1 = _scat_call(g1, src4, dst4)
    h2, g2 = _mid_call1(s1, g1, dinv, b1r, gm1, bt1, h1, W2)
    s2 = _scat_call(g2, src4, dst4)
    return _fin_call(s2, g2, dinv, b2r, gm2, bt2, h2)


# speedup vs baseline: 1.2428x; 1.0820x over previous
import jax
import jax.numpy as jnp
from jax import lax
from jax.experimental import pallas as pl
from jax.experimental.pallas import tpu as pltpu
from jax.experimental.pallas import tpu_sc as plsc

N = 10000
D = 128
NC = 2
NS = 16
CH = 128
K0 = 96
K1 = 61
KM = max(K0, K1)
E0 = NS * K0 * CH
E1CAP = NS * K1 * CH
NPAD = 10112
RPT = NPAD // NS
DEGW = 16

_mesh = plsc.VectorSubcoreMesh(core_axis_name="c", subcore_axis_name="s",
                               num_cores=NC, num_subcores=NS)


def _deg_body(dst_hbm, degw_hbm, dstv, onesv, zv, acc):
    cid = lax.axis_index("c")
    tid = lax.axis_index("s")
    pltpu.sync_copy(dst_hbm.at[cid, tid], dstv)

    def fill(i, _):
        onesv[i, :] = jnp.full((DEGW,), 1.0, jnp.float32)
        zv[i, :] = jnp.zeros((DEGW,), jnp.float32)
        return 0

    lax.fori_loop(0, CH, fill, 0)
    base = tid * RPT
    for r0 in range(0, RPT, CH):
        sz = min(CH, RPT - r0)
        pltpu.sync_copy(zv.at[pl.ds(0, sz)], acc.at[pl.ds(base + r0, sz)])
    plsc.subcore_barrier()

    def chunk(j, _):
        pltpu.sync_copy(onesv, acc.at[dstv.at[j]], add=True)
        return 0

    myk = jnp.where(cid == 0, K0, K1)
    lax.fori_loop(0, myk, chunk, 0)
    plsc.subcore_barrier()
    pltpu.sync_copy(acc.at[pl.ds(base, RPT)], degw_hbm.at[cid, pl.ds(base, RPT)])


_deg_call = pl.kernel(
    _deg_body,
    out_type=jax.ShapeDtypeStruct((NC, NPAD, DEGW), jnp.float32),
    mesh=_mesh,
    scratch_types=[
        pltpu.VMEM((KM, CH), jnp.int32),
        pltpu.VMEM((CH, DEGW), jnp.float32),
        pltpu.VMEM((CH, DEGW), jnp.float32),
        pltpu.VMEM_SHARED((NPAD, DEGW), jnp.float32),
    ],
)


def _scat_body(g_hbm, src_hbm, dst_hbm, s_hbm, srcv, dstv, rows, acc):
    cid = lax.axis_index("c")
    tid = lax.axis_index("s")
    pltpu.sync_copy(src_hbm.at[cid, tid], srcv)
    pltpu.sync_copy(dst_hbm.at[cid, tid], dstv)

    def zb(k, _):
        rows[k // 8, pl.ds((k % 8) * 16, 16)] = jnp.zeros((16,), jnp.float32)
        return 0

    lax.fori_loop(0, CH * D // 16, zb, 0)
    base = tid * RPT
    for r0 in range(0, RPT, CH):
        sz = min(CH, RPT - r0)
        pltpu.sync_copy(rows.at[pl.ds(0, sz)], acc.at[pl.ds(base + r0, sz)])
    plsc.subcore_barrier()

    def chunk(j, _):
        pltpu.sync_copy(g_hbm.at[srcv.at[j]], rows)
        pltpu.sync_copy(rows, acc.at[dstv.at[j]], add=True)
        return 0

    myk = jnp.where(cid == 0, K0, K1)
    lax.fori_loop(0, myk, chunk, 0)
    plsc.subcore_barrier()
    pltpu.sync_copy(acc.at[pl.ds(base, RPT)], s_hbm.at[cid, pl.ds(base, RPT)])


_scat_call = pl.kernel(
    _scat_body,
    out_type=jax.ShapeDtypeStruct((NC, NPAD, D), jnp.float32),
    mesh=_mesh,
    scratch_types=[
        pltpu.VMEM((KM, CH), jnp.int32),
        pltpu.VMEM((KM, CH), jnp.int32),
        pltpu.VMEM((CH, D), jnp.float32),
        pltpu.VMEM_SHARED((NPAD, D), jnp.float32),
    ],
)


def _pre_body(degw_ref, x_ref, w_ref, dinv_ref, g_ref):
    t = degw_ref[...]
    deg = t[0, 0:N, 0:1] + t[1, 0:N, 0:1] + 1.0
    dinv = 1.0 / jnp.sqrt(deg)
    dinv_ref[...] = dinv
    g_ref[...] = dinv * jnp.dot(x_ref[...], w_ref[...],
                                preferred_element_type=jnp.float32)


def _mid_body(s_ref, g_ref, dinv_ref, b_ref, gm_ref, bt_ref, w_ref, h_ref,
              gn_ref, hp_ref=None):
    g = g_ref[...]
    s = s_ref[...]
    dinv = dinv_ref[...]
    o = dinv * (s[0, 0:N, :] + s[1, 0:N, :] + g) + b_ref[...]
    mu = jnp.mean(o, axis=0, keepdims=True)
    xc = o - mu
    var = jnp.mean(xc * xc, axis=0, keepdims=True)
    bn = gm_ref[...] * xc / jnp.sqrt(var + 1e-5) + bt_ref[...]
    h = jnp.maximum(bn, 0.0)
    if hp_ref is not None:
        h = hp_ref[...] + h
    h_ref[...] = h
    gn_ref[...] = dinv * jnp.dot(h, w_ref[...],
                                 preferred_element_type=jnp.float32)


def _mid_body_resid(s_ref, g_ref, dinv_ref, b_ref, gm_ref, bt_ref, hp_ref,
                    w_ref, h_ref, gn_ref):
    _mid_body(s_ref, g_ref, dinv_ref, b_ref, gm_ref, bt_ref, w_ref, h_ref,
              gn_ref, hp_ref=hp_ref)


def _fin_body(s_ref, g_ref, dinv_ref, b_ref, gm_ref, bt_ref, hp_ref, out_ref):
    g = g_ref[...]
    s = s_ref[...]
    o = dinv_ref[...] * (s[0, 0:N, :] + s[1, 0:N, :] + g) + b_ref[...]
    mu = jnp.mean(o, axis=0, keepdims=True)
    xc = o - mu
    var = jnp.mean(xc * xc, axis=0, keepdims=True)
    bn = gm_ref[...] * xc / jnp.sqrt(var + 1e-5) + bt_ref[...]
    out_ref[...] = hp_ref[...] + bn


_pre_call = pl.pallas_call(
    _pre_body,
    out_shape=[jax.ShapeDtypeStruct((N, 1), jnp.float32),
               jax.ShapeDtypeStruct((N, D), jnp.float32)],
)

_mid_call0 = pl.pallas_call(
    _mid_body,
    out_shape=[jax.ShapeDtypeStruct((N, D), jnp.float32),
               jax.ShapeDtypeStruct((N, D), jnp.float32)],
)

_mid_call1 = pl.pallas_call(
    _mid_body_resid,
    out_shape=[jax.ShapeDtypeStruct((N, D), jnp.float32),
               jax.ShapeDtypeStruct((N, D), jnp.float32)],
)

_fin_call = pl.pallas_call(
    _fin_body,
    out_shape=jax.ShapeDtypeStruct((N, D), jnp.float32),
)


def kernel(x, edge_index, W0, b0, gamma0, beta0, W1, b1, gamma1, beta1,
           W2, b2, gamma2, beta2):
    src = edge_index[0]
    dst = edge_index[1]
    e = src.shape[0]
    pad1 = E1CAP - (e - E0)
    padm = NS * (KM - K1) * CH
    sp = NPAD - N
    dpad1 = N + (jnp.arange(pad1, dtype=jnp.int32) % sp)
    dpadm = N + (jnp.arange(padm, dtype=jnp.int32) % sp)
    src0 = src[:E0].reshape(NS, K0, CH)
    dst0 = dst[:E0].reshape(NS, K0, CH)
    src1 = jnp.concatenate(
        [jnp.concatenate([src[E0:], jnp.zeros((pad1,), jnp.int32)]
                         ).reshape(NS, K1, CH),
         jnp.zeros((padm,), jnp.int32).reshape(NS, KM - K1, CH)], axis=1)
    dst1 = jnp.concatenate(
        [jnp.concatenate([dst[E0:], dpad1]).reshape(NS, K1, CH),
         dpadm.reshape(NS, KM - K1, CH)], axis=1)
    src4 = jnp.stack([src0, src1])
    dst4 = jnp.stack([dst0, dst1])
    b0r, b1r, b2r = (v.reshape(1, D) for v in (b0, b1, b2))
    gm0, gm1, gm2 = (v.reshape(1, D) for v in (gamma0, gamma1, gamma2))
    bt0, bt1, bt2 = (v.reshape(1, D) for v in (beta0, beta1, beta2))

    degw = _deg_call(dst4)
    dinv, g0 = _pre_call(degw, x, W0)
    s0 = _scat_call(g0, src4, dst4)
    h1, g1 = _mid_call0(s0, g0, dinv, b0r, gm0, bt0, W1)
    s1 = _scat_call(g1, src4, dst4)
    h2, g2 = _mid_call1(s1, g1, dinv, b1r, gm1, bt1, h1, W2)
    s2 = _scat_call(g2, src4, dst4)
    return _fin_call(s2, g2, dinv, b2r, gm2, bt2, h2)
